# baseline (speedup 1.0000x reference)
import math

import jax
import jax.numpy as jnp
import numpy as np
from jax.experimental import pallas as pl

_THETA = 2.0
_CX = 300
_CY = 250
_LINE_WIDTH = 25
_COLOR = np.array([0.37, 0.12, 0.88], dtype=np.float32) * 0.2
_MASK_FILL = np.array([0.5, 0.5, 0.0], dtype=np.float32)

_H = 512
_W = 512
_LINE_LEN_I = int(math.hypot(_H, _W))
_DX = int(math.cos(_THETA) * _LINE_LEN_I)
_DY = int(math.sin(_THETA) * _LINE_LEN_I)
_X0 = _CX - _DX // 2
_Y0 = _CY - _DY // 2
_X1 = _CX + _DX // 2
_Y1 = _CY + _DY // 2

_BH = 64


def _line_kernel(img_ref, mask_ref, img_out_ref, mask_out_ref):
    i = pl.program_id(0)
    y0f = jnp.float32(_Y0)
    x0f = jnp.float32(_X0)
    vx = jnp.float32(_X1 - _X0)
    vy = jnp.float32(_Y1 - _Y0)
    line_len = jnp.sqrt(vx * vx + vy * vy)

    yy = jax.lax.broadcasted_iota(jnp.int32, (_BH, _W), 0)
    xx = jax.lax.broadcasted_iota(jnp.int32, (_BH, _W), 1)
    py = (yy + (i * _BH - _Y0)).astype(jnp.float32)
    px = xx.astype(jnp.float32) - x0f
    cross = vy * px - vx * py
    dist = jnp.abs(cross) / line_len
    dot = (px * vx + py * vy) / (line_len * line_len)
    band = (dist <= jnp.float32(_LINE_WIDTH / 2)) & (dot >= 0) & (dot <= 1)
    for c in range(3):
        img_out_ref[c] = jnp.where(band, jnp.float32(_COLOR[c]), img_ref[c])
        mask_out_ref[c] = jnp.where(band, jnp.float32(_MASK_FILL[c]), mask_ref[c])


def kernel(img, mask):
    C, H, W = img.shape
    spec = pl.BlockSpec((C, _BH, W), lambda i: (0, i, 0))
    img_out, mask_out = pl.pallas_call(
        _line_kernel,
        grid=(H // _BH,),
        in_specs=[spec, spec],
        out_specs=[spec, spec],
        out_shape=[
            jax.ShapeDtypeStruct(img.shape, img.dtype),
            jax.ShapeDtypeStruct(mask.shape, mask.dtype),
        ],
    )(img, mask)
    return (img_out, mask_out)


# speedup vs baseline: 1.5859x; 1.5859x over previous
import math

import jax
import jax.numpy as jnp
import numpy as np
from jax.experimental import pallas as pl

_THETA = 2.0
_CX = 300
_CY = 250
_LINE_WIDTH = 25
_COLOR = np.array([0.37, 0.12, 0.88], dtype=np.float32) * 0.2
_MASK_FILL = np.array([0.5, 0.5, 0.0], dtype=np.float32)

_H = 512
_W = 512
_LINE_LEN_I = int(math.hypot(_H, _W))
_DX = int(math.cos(_THETA) * _LINE_LEN_I)
_DY = int(math.sin(_THETA) * _LINE_LEN_I)
_X0 = _CX - _DX // 2
_Y0 = _CY - _DY // 2
_X1 = _CX + _DX // 2
_Y1 = _CY + _DY // 2

_BH = 256


def _line_kernel(img_ref, mask_ref, img_out_ref, mask_out_ref):
    i = pl.program_id(0)
    y0f = jnp.float32(_Y0)
    x0f = jnp.float32(_X0)
    vx = jnp.float32(_X1 - _X0)
    vy = jnp.float32(_Y1 - _Y0)
    line_len = jnp.sqrt(vx * vx + vy * vy)

    yy = jax.lax.broadcasted_iota(jnp.int32, (_BH, _W), 0)
    xx = jax.lax.broadcasted_iota(jnp.int32, (_BH, _W), 1)
    py = (yy + (i * _BH - _Y0)).astype(jnp.float32)
    px = xx.astype(jnp.float32) - x0f
    cross = vy * px - vx * py
    dist = jnp.abs(cross) / line_len
    dot = (px * vx + py * vy) / (line_len * line_len)
    band = (dist <= jnp.float32(_LINE_WIDTH / 2)) & (dot >= 0) & (dot <= 1)
    for c in range(3):
        img_out_ref[c] = jnp.where(band, jnp.float32(_COLOR[c]), img_ref[c])
        mask_out_ref[c] = jnp.where(band, jnp.float32(_MASK_FILL[c]), mask_ref[c])


def kernel(img, mask):
    C, H, W = img.shape
    spec = pl.BlockSpec((C, _BH, W), lambda i: (0, i, 0))
    img_out, mask_out = pl.pallas_call(
        _line_kernel,
        grid=(H // _BH,),
        in_specs=[spec, spec],
        out_specs=[spec, spec],
        out_shape=[
            jax.ShapeDtypeStruct(img.shape, img.dtype),
            jax.ShapeDtypeStruct(mask.shape, mask.dtype),
        ],
    )(img, mask)
    return (img_out, mask_out)
